# baseline (speedup 1.0000x reference)
import functools

import jax
import jax.numpy as jnp
from jax import lax
from jax.experimental import pallas as pl
from jax.experimental.pallas import tpu as pltpu
from jax.experimental.pallas import tpu_sc as plsc

B, N, P = 16, 2048, 256
M, L = 512, 9

NC = 2
NS = 16
LANES = 16
NW = NC * NS

ROWS = M * P
RPW = ROWS // NW
CH = 256
NCHUNK = RPW // CH
GIDX = 128
GB = CH * L // GIDX
IDX_BLOCKS = ROWS * L // GIDX


NG = 16


def _tc_pack_body(x_ref, o_ref):
    xb = x_ref[...]
    t = jnp.transpose(xb, (1, 2, 0))
    t3 = t.reshape(NG, P // 8, 8, B)
    o_ref[...] = jnp.concatenate([t3[:, :, j, :] for j in range(8)], axis=-1)


def _tc_pack(x):
    return pl.pallas_call(
        _tc_pack_body,
        grid=(N // NG,),
        in_specs=[pl.BlockSpec((B, NG, P), lambda i: (0, i, 0))],
        out_specs=pl.BlockSpec((NG, P // 8, 8 * B), lambda i: (i, 0, 0)),
        out_shape=jax.ShapeDtypeStruct((N, P // 8, 8 * B), jnp.float32),
    )(x)


def _sc_gather_max(xt, idx_blocks):
    mesh = plsc.VectorSubcoreMesh(core_axis_name="c", subcore_axis_name="s")

    @functools.partial(
        pl.kernel,
        mesh=mesh,
        compiler_params=pltpu.CompilerParams(use_tc_tiling_on_sc=False),
        out_type=jax.ShapeDtypeStruct((ROWS * LANES,), jnp.float32),
        scratch_types=[
            pltpu.VMEM((RPW * L // GIDX, GIDX), jnp.int32),
            pltpu.VMEM((2 * CH * L, LANES), jnp.float32),
            pltpu.VMEM((2 * CH * LANES,), jnp.float32),
            pltpu.SemaphoreType.DMA,
            pltpu.SemaphoreType.DMA,
            pltpu.SemaphoreType.DMA,
            pltpu.SemaphoreType.DMA,
        ],
    )
    def k(xt_hbm, idx_hbm, out_hbm, idx_v, rows_v, out_v,
          gsem0, gsem1, osem0, osem1):
        wid = lax.axis_index("s") * NC + lax.axis_index("c")
        gsems = (gsem0, gsem1)
        osems = (osem0, osem1)
        blk_per_w = RPW * L // GIDX
        pltpu.sync_copy(idx_hbm.at[pl.ds(wid * blk_per_w, blk_per_w), :], idx_v)

        def fire(c):
            par = c % 2
            for j in range(GB):
                pltpu.async_copy(
                    xt_hbm.at[idx_v.at[c * GB + j]],
                    rows_v.at[pl.ds(par * CH * L + j * GIDX, GIDX), :],
                    gsems[par],
                )

        def drain(c):
            par = c % 2
            for j in range(GB):
                pltpu.make_async_copy(
                    xt_hbm.at[idx_v.at[c * GB + j]],
                    rows_v.at[pl.ds(par * CH * L + j * GIDX, GIDX), :],
                    gsems[par],
                ).wait()

        fire(0)
        for c in range(NCHUNK):
            par = c % 2
            if c + 1 < NCHUNK:
                fire(c + 1)
            if c >= 2:
                pltpu.make_async_copy(
                    out_v.at[pl.ds(par * CH * LANES, CH * LANES)],
                    out_hbm.at[pl.ds((wid * RPW + (c - 2) * CH) * LANES,
                                     CH * LANES)],
                    osems[par],
                ).wait()
            drain(c)

            def row_body(r, carry2, _par=par):
                base = _par * CH * L + r * L
                v = rows_v[base]
                for l in range(1, L):
                    v = jnp.maximum(v, rows_v[base + l])
                out_v[pl.ds((_par * CH + r) * LANES, LANES)] = v
                return carry2

            lax.fori_loop(0, CH, row_body, 0, unroll=2)
            pltpu.async_copy(
                out_v.at[pl.ds(par * CH * LANES, CH * LANES)],
                out_hbm.at[pl.ds((wid * RPW + c * CH) * LANES, CH * LANES)],
                osems[par],
            )
        for c in (NCHUNK - 2, NCHUNK - 1):
            par = c % 2
            pltpu.make_async_copy(
                out_v.at[pl.ds(par * CH * LANES, CH * LANES)],
                out_hbm.at[pl.ds((wid * RPW + c * CH) * LANES, CH * LANES)],
                osems[par],
            ).wait()

    return k(xt, idx_blocks)


def kernel(x, LRF_getter):
    xt = _tc_pack(x).reshape(N * P, B)
    idx_n = LRF_getter[..., 0]
    idx_c = LRF_getter[..., 1]
    flat = (idx_n * P + idx_c).reshape(IDX_BLOCKS, GIDX)
    out_t = _sc_gather_max(xt, flat)
    return jnp.transpose(out_t.reshape(M, P, B), (2, 0, 1))


# speedup vs baseline: 1.2642x; 1.2642x over previous
import functools

import jax
import jax.numpy as jnp
from jax import lax
from jax.experimental import pallas as pl
from jax.experimental.pallas import tpu as pltpu
from jax.experimental.pallas import tpu_sc as plsc

B, N, P = 16, 2048, 256
M, L = 512, 9

NC = 2
NS = 16
LANES = 16
NW = NC * NS

ROWS = M * P
RPW = ROWS // NW
CH = 256
NCHUNK = RPW // CH
GIDX = 128
GB = CH * L // GIDX
IDX_BLOCKS = ROWS * L // GIDX


GPW = N // 8 // NW
NSLAB = B * 2048


def _sc_transpose(xx2):
    mesh = plsc.VectorSubcoreMesh(core_axis_name="c", subcore_axis_name="s")

    @functools.partial(
        pl.kernel,
        mesh=mesh,
        compiler_params=pltpu.CompilerParams(use_tc_tiling_on_sc=False),
        out_type=jax.ShapeDtypeStruct((N * P * B,), jnp.float32),
        scratch_types=[
            pltpu.VMEM((2 * B, 2048), jnp.float32),
            pltpu.VMEM((2 * P * B,), jnp.float32),
            pltpu.SemaphoreType.DMA,
            pltpu.SemaphoreType.DMA,
            pltpu.SemaphoreType.DMA,
            pltpu.SemaphoreType.DMA,
        ],
    )
    def k(xx_hbm, xt_hbm, slab_v, out_v, ssem0, ssem1, osem0, osem1):
        wid = lax.axis_index("s") * NC + lax.axis_index("c")
        ssems = (ssem0, ssem1)
        osems = (osem0, osem1)
        iota16 = lax.iota(jnp.int32, 16)

        def slab_copy(gi):
            par = gi % 2
            return pltpu.make_async_copy(
                xx_hbm.at[:, wid * GPW + gi, :],
                slab_v.at[pl.ds(par * B, B), :],
                ssems[par],
            )

        slab_copy(0).start()
        for gi in range(GPW):
            par = gi % 2
            if gi + 1 < GPW:
                slab_copy(gi + 1).start()
            slab_copy(gi).wait()
            row_idx = iota16 + par * B
            for ns in range(8):
                k_ns = gi * 8 + ns
                opar = k_ns % 2
                n_out = wid * GPW * 8 + k_ns
                if k_ns >= 2:
                    pltpu.make_async_copy(
                        out_v.at[pl.ds(opar * P * B, P * B)],
                        xt_hbm.at[pl.ds((n_out - 2) * P * B, P * B)],
                        osems[opar],
                    ).wait()
                for pt in range(2):

                    def p_body(r, col, _opar=opar, _pt=pt):
                        v = plsc.load_gather(slab_v, [row_idx, col])
                        out_v[pl.ds((_opar * P + _pt * 128 + r) * B, B)] = v
                        return col + 1

                    col0 = jnp.full((16,), pt * 1024 + ns * 128, jnp.int32)
                    lax.fori_loop(0, 128, p_body, col0, unroll=4)
                pltpu.async_copy(
                    out_v.at[pl.ds(opar * P * B, P * B)],
                    xt_hbm.at[pl.ds(n_out * P * B, P * B)],
                    osems[opar],
                )
        for k_ns in (GPW * 8 - 2, GPW * 8 - 1):
            opar = k_ns % 2
            n_out = wid * GPW * 8 + k_ns
            pltpu.make_async_copy(
                out_v.at[pl.ds(opar * P * B, P * B)],
                xt_hbm.at[pl.ds(n_out * P * B, P * B)],
                osems[opar],
            ).wait()

    return k(xx2)


def _sc_gather_max(xt, idx_blocks):
    mesh = plsc.VectorSubcoreMesh(core_axis_name="c", subcore_axis_name="s")

    @functools.partial(
        pl.kernel,
        mesh=mesh,
        compiler_params=pltpu.CompilerParams(use_tc_tiling_on_sc=False),
        out_type=jax.ShapeDtypeStruct((ROWS * LANES,), jnp.float32),
        scratch_types=[
            pltpu.VMEM((RPW * L // GIDX, GIDX), jnp.int32),
            pltpu.VMEM((2 * CH * L, LANES), jnp.float32),
            pltpu.VMEM((2 * CH * LANES,), jnp.float32),
            pltpu.SemaphoreType.DMA,
            pltpu.SemaphoreType.DMA,
            pltpu.SemaphoreType.DMA,
            pltpu.SemaphoreType.DMA,
        ],
    )
    def k(xt_hbm, idx_hbm, out_hbm, idx_v, rows_v, out_v,
          gsem0, gsem1, osem0, osem1):
        wid = lax.axis_index("s") * NC + lax.axis_index("c")
        gsems = (gsem0, gsem1)
        osems = (osem0, osem1)
        blk_per_w = RPW * L // GIDX
        pltpu.sync_copy(idx_hbm.at[pl.ds(wid * blk_per_w, blk_per_w), :], idx_v)

        def fire(c):
            par = c % 2
            for j in range(GB):
                pltpu.async_copy(
                    xt_hbm.at[idx_v.at[c * GB + j]],
                    rows_v.at[pl.ds(par * CH * L + j * GIDX, GIDX), :],
                    gsems[par],
                )

        def drain(c):
            par = c % 2
            for j in range(GB):
                pltpu.make_async_copy(
                    xt_hbm.at[idx_v.at[c * GB + j]],
                    rows_v.at[pl.ds(par * CH * L + j * GIDX, GIDX), :],
                    gsems[par],
                ).wait()

        fire(0)
        for c in range(NCHUNK):
            par = c % 2
            if c + 1 < NCHUNK:
                fire(c + 1)
            if c >= 2:
                pltpu.make_async_copy(
                    out_v.at[pl.ds(par * CH * LANES, CH * LANES)],
                    out_hbm.at[pl.ds((wid * RPW + (c - 2) * CH) * LANES,
                                     CH * LANES)],
                    osems[par],
                ).wait()
            drain(c)

            def row_body(r, carry2, _par=par):
                base = _par * CH * L + r * L
                v = rows_v[base]
                for l in range(1, L):
                    v = jnp.maximum(v, rows_v[base + l])
                out_v[pl.ds((_par * CH + r) * LANES, LANES)] = v
                return carry2

            lax.fori_loop(0, CH, row_body, 0, unroll=2)
            pltpu.async_copy(
                out_v.at[pl.ds(par * CH * LANES, CH * LANES)],
                out_hbm.at[pl.ds((wid * RPW + c * CH) * LANES, CH * LANES)],
                osems[par],
            )
        for c in (NCHUNK - 2, NCHUNK - 1):
            par = c % 2
            pltpu.make_async_copy(
                out_v.at[pl.ds(par * CH * LANES, CH * LANES)],
                out_hbm.at[pl.ds((wid * RPW + c * CH) * LANES, CH * LANES)],
                osems[par],
            ).wait()

    return k(xt, idx_blocks)


def kernel(x, LRF_getter):
    xt = jnp.transpose(x, (1, 2, 0)).reshape(N * P, B)
    idx_n = LRF_getter[..., 0]
    idx_c = LRF_getter[..., 1]
    flat = (idx_n * P + idx_c).reshape(IDX_BLOCKS, GIDX)
    out_t = _sc_gather_max(xt, flat)
    return jnp.transpose(out_t.reshape(M, P, B), (2, 0, 1))


# speedup vs baseline: 1.3101x; 1.0363x over previous
import functools

import jax
import jax.numpy as jnp
from jax import lax
from jax.experimental import pallas as pl
from jax.experimental.pallas import tpu as pltpu
from jax.experimental.pallas import tpu_sc as plsc

B, N, P = 16, 2048, 256
M, L = 512, 9

NC = 2
NS = 16
LANES = 16
NW = NC * NS

ROWS = M * P
RPW = ROWS // NW
CH = 256
NCHUNK = RPW // CH
GIDX = 128
GB = CH * L // GIDX
IDX_BLOCKS = ROWS * L // GIDX


GPW = N // 8 // NW
NSLAB = B * 2048

NG = 16


PCOLS = NG * P // 8


def _tc_pack_body(x_ref, o_ref):
    xb = x_ref[...]
    parts = [
        jnp.transpose(xb[:, j * PCOLS:(j + 1) * PCOLS], (1, 0))
        for j in range(8)
    ]
    o_ref[...] = jnp.concatenate(parts, axis=1)


def _tc_pack(x):
    return pl.pallas_call(
        _tc_pack_body,
        grid=(N // NG,),
        in_specs=[pl.BlockSpec((B, NG * P), lambda i: (0, i))],
        out_specs=pl.BlockSpec((NG * P // 8, 8 * B), lambda i: (i, 0)),
        out_shape=jax.ShapeDtypeStruct((N * P // 8, 8 * B), jnp.float32),
    )(x.reshape(B, N * P))


def _sc_transpose(xx2):
    mesh = plsc.VectorSubcoreMesh(core_axis_name="c", subcore_axis_name="s")

    @functools.partial(
        pl.kernel,
        mesh=mesh,
        compiler_params=pltpu.CompilerParams(use_tc_tiling_on_sc=False),
        out_type=jax.ShapeDtypeStruct((N * P * B,), jnp.float32),
        scratch_types=[
            pltpu.VMEM((2 * B, 2048), jnp.float32),
            pltpu.VMEM((2 * P * B,), jnp.float32),
            pltpu.SemaphoreType.DMA,
            pltpu.SemaphoreType.DMA,
            pltpu.SemaphoreType.DMA,
            pltpu.SemaphoreType.DMA,
        ],
    )
    def k(xx_hbm, xt_hbm, slab_v, out_v, ssem0, ssem1, osem0, osem1):
        wid = lax.axis_index("s") * NC + lax.axis_index("c")
        ssems = (ssem0, ssem1)
        osems = (osem0, osem1)
        iota16 = lax.iota(jnp.int32, 16)

        def slab_copy(gi):
            par = gi % 2
            return pltpu.make_async_copy(
                xx_hbm.at[:, wid * GPW + gi, :],
                slab_v.at[pl.ds(par * B, B), :],
                ssems[par],
            )

        slab_copy(0).start()
        for gi in range(GPW):
            par = gi % 2
            if gi + 1 < GPW:
                slab_copy(gi + 1).start()
            slab_copy(gi).wait()
            row_idx = iota16 + par * B
            for ns in range(8):
                k_ns = gi * 8 + ns
                opar = k_ns % 2
                n_out = wid * GPW * 8 + k_ns
                if k_ns >= 2:
                    pltpu.make_async_copy(
                        out_v.at[pl.ds(opar * P * B, P * B)],
                        xt_hbm.at[pl.ds((n_out - 2) * P * B, P * B)],
                        osems[opar],
                    ).wait()
                for pt in range(2):

                    def p_body(r, col, _opar=opar, _pt=pt):
                        v = plsc.load_gather(slab_v, [row_idx, col])
                        out_v[pl.ds((_opar * P + _pt * 128 + r) * B, B)] = v
                        return col + 1

                    col0 = jnp.full((16,), pt * 1024 + ns * 128, jnp.int32)
                    lax.fori_loop(0, 128, p_body, col0, unroll=4)
                pltpu.async_copy(
                    out_v.at[pl.ds(opar * P * B, P * B)],
                    xt_hbm.at[pl.ds(n_out * P * B, P * B)],
                    osems[opar],
                )
        for k_ns in (GPW * 8 - 2, GPW * 8 - 1):
            opar = k_ns % 2
            n_out = wid * GPW * 8 + k_ns
            pltpu.make_async_copy(
                out_v.at[pl.ds(opar * P * B, P * B)],
                xt_hbm.at[pl.ds(n_out * P * B, P * B)],
                osems[opar],
            ).wait()

    return k(xx2)


def _sc_gather_max(xt, idx_blocks):
    mesh = plsc.VectorSubcoreMesh(core_axis_name="c", subcore_axis_name="s")

    @functools.partial(
        pl.kernel,
        mesh=mesh,
        compiler_params=pltpu.CompilerParams(use_tc_tiling_on_sc=False),
        out_type=jax.ShapeDtypeStruct((ROWS * LANES,), jnp.float32),
        scratch_types=[
            pltpu.VMEM((RPW * L // GIDX, GIDX), jnp.int32),
            pltpu.VMEM((2 * CH * L, LANES), jnp.float32),
            pltpu.VMEM((2 * CH * LANES,), jnp.float32),
            pltpu.SemaphoreType.DMA,
            pltpu.SemaphoreType.DMA,
            pltpu.SemaphoreType.DMA,
            pltpu.SemaphoreType.DMA,
        ],
    )
    def k(xt_hbm, idx_hbm, out_hbm, idx_v, rows_v, out_v,
          gsem0, gsem1, osem0, osem1):
        wid = lax.axis_index("s") * NC + lax.axis_index("c")
        gsems = (gsem0, gsem1)
        osems = (osem0, osem1)
        blk_per_w = RPW * L // GIDX
        pltpu.sync_copy(idx_hbm.at[pl.ds(wid * blk_per_w, blk_per_w), :], idx_v)

        def fire(c):
            par = c % 2
            for j in range(GB):
                pltpu.async_copy(
                    xt_hbm.at[idx_v.at[c * GB + j]],
                    rows_v.at[pl.ds(par * CH * L + j * GIDX, GIDX), :],
                    gsems[par],
                )

        def drain(c):
            par = c % 2
            for j in range(GB):
                pltpu.make_async_copy(
                    xt_hbm.at[idx_v.at[c * GB + j]],
                    rows_v.at[pl.ds(par * CH * L + j * GIDX, GIDX), :],
                    gsems[par],
                ).wait()

        fire(0)
        for c in range(NCHUNK):
            par = c % 2
            if c + 1 < NCHUNK:
                fire(c + 1)
            if c >= 2:
                pltpu.make_async_copy(
                    out_v.at[pl.ds(par * CH * LANES, CH * LANES)],
                    out_hbm.at[pl.ds((wid * RPW + (c - 2) * CH) * LANES,
                                     CH * LANES)],
                    osems[par],
                ).wait()
            drain(c)

            def row_body(r, carry2, _par=par):
                base = _par * CH * L + r * L
                v = rows_v[base]
                for l in range(1, L):
                    v = jnp.maximum(v, rows_v[base + l])
                out_v[pl.ds((_par * CH + r) * LANES, LANES)] = v
                return carry2

            lax.fori_loop(0, CH, row_body, 0, unroll=2)
            pltpu.async_copy(
                out_v.at[pl.ds(par * CH * LANES, CH * LANES)],
                out_hbm.at[pl.ds((wid * RPW + c * CH) * LANES, CH * LANES)],
                osems[par],
            )
        for c in (NCHUNK - 2, NCHUNK - 1):
            par = c % 2
            pltpu.make_async_copy(
                out_v.at[pl.ds(par * CH * LANES, CH * LANES)],
                out_hbm.at[pl.ds((wid * RPW + c * CH) * LANES, CH * LANES)],
                osems[par],
            ).wait()

    return k(xt, idx_blocks)


def kernel(x, LRF_getter):
    xt = _tc_pack(x).reshape(N * P, B)
    idx_n = LRF_getter[..., 0]
    idx_c = LRF_getter[..., 1]
    q = idx_n * P + idx_c
    g = (q & ~(NG * P - 1)) + (q % PCOLS) * 8 + (q % (NG * P)) // PCOLS
    flat = g.reshape(IDX_BLOCKS, GIDX)
    out_t = _sc_gather_max(xt, flat)
    return jnp.transpose(out_t.reshape(M, P, B), (2, 0, 1))
